# baseline (speedup 1.0000x reference)
import jax
import jax.numpy as jnp
from jax.experimental import pallas as pl
from jax.experimental.pallas import tpu as pltpu


BN = 8192
NC = 4
CH = BN // NC


def _body(x_ref, w_ref, b_ref, o_hbm, o_vmem, sems):
    i = pl.program_id(0)
    bias = jnp.reshape(b_ref[...], (b_ref.shape[0], 1))
    w = w_ref[...]
    for c in range(NC):
        o_vmem[:, pl.ds(c * CH, CH)] = (
            jnp.dot(w, x_ref[:, pl.ds(c * CH, CH)], preferred_element_type=jnp.float32)
            + bias
        )
        pltpu.make_async_copy(
            o_vmem.at[:, pl.ds(c * CH, CH)],
            o_hbm.at[:, pl.ds(i * BN + c * CH, CH)],
            sems.at[c],
        ).start()
    for c in range(NC):
        pltpu.make_async_copy(
            o_vmem.at[:, pl.ds(c * CH, CH)],
            o_hbm.at[:, pl.ds(i * BN + c * CH, CH)],
            sems.at[c],
        ).wait()


def kernel(scalar, W, b):
    batch, k = scalar.shape
    n = W.shape[1]
    xT = scalar.T
    wT = W.T
    grid = batch // BN
    outT = pl.pallas_call(
        _body,
        grid=(grid,),
        in_specs=[
            pl.BlockSpec((k, BN), lambda i: (0, i)),
            pl.BlockSpec((n, k), lambda i: (0, 0)),
            pl.BlockSpec((n,), lambda i: (0,)),
        ],
        out_specs=pl.BlockSpec(memory_space=pl.ANY),
        out_shape=jax.ShapeDtypeStruct((n, batch), jnp.float32),
        scratch_shapes=[
            pltpu.MemorySpace.VMEM((n, BN), jnp.float32),
            pltpu.SemaphoreType.DMA((NC,)),
        ],
    )(xT, wT, b)
    return outT.T


# speedup vs baseline: 1.0362x; 1.0362x over previous
import jax
import jax.numpy as jnp
from jax.experimental import pallas as pl


BN = 8192


def _body(x_ref, w_ref, b_ref, o_ref):
    bias = jnp.reshape(b_ref[...], (b_ref.shape[0], 1))
    o_ref[...] = (
        jnp.dot(w_ref[...], x_ref[...], preferred_element_type=jnp.float32)
        + bias
    )


def kernel(scalar, W, b):
    batch, k = scalar.shape
    n = W.shape[1]
    xT = scalar.T
    wT = W.T
    grid = batch // BN
    outT = pl.pallas_call(
        _body,
        grid=(grid,),
        in_specs=[
            pl.BlockSpec((k, BN), lambda i: (0, i)),
            pl.BlockSpec((n, k), lambda i: (0, 0)),
            pl.BlockSpec((n,), lambda i: (0,)),
        ],
        out_specs=pl.BlockSpec((n, BN), lambda i: (0, i)),
        out_shape=jax.ShapeDtypeStruct((n, batch), jnp.float32),
    )(xT, wT, b)
    return outT.T
